# baseline (speedup 1.0000x reference)
import functools

import jax
import jax.numpy as jnp
from jax import lax
from jax.experimental import pallas as pl
from jax.experimental.pallas import tpu as pltpu
from jax.experimental.pallas import tpu_sc as plsc


def _sc_gather(node_tab, edge_tab, plist_flat, B, N, H, M):
    info = plsc.get_sparse_core_info()
    NC, NS, L = info.num_cores, info.num_subcores, info.num_lanes
    NW = NC * NS
    PW = M // NW
    CH = 64
    NCH = PW // CH
    WPB = NW // B

    mesh = plsc.VectorSubcoreMesh(core_axis_name="c", subcore_axis_name="s")

    @functools.partial(
        pl.kernel,
        mesh=mesh,
        compiler_params=pltpu.CompilerParams(use_tc_tiling_on_sc=False,
                                             needs_layout_passes=False),
        out_type=(
            jax.ShapeDtypeStruct((2 * M, H), jnp.float32),
            jax.ShapeDtypeStruct((M, 2 * H), jnp.float32),
        ),
        scratch_types=[
            pltpu.VMEM((PW,), jnp.int32),
            pltpu.VMEM((PW,), jnp.int32),
            pltpu.VMEM((2 * PW,), jnp.int32),
            pltpu.VMEM((PW,), jnp.int32),
            pltpu.VMEM((2 * CH, H), jnp.float32),
            pltpu.VMEM((CH, 2 * H), jnp.float32),
            pltpu.SemaphoreType.DMA,
        ],
    )
    def gather_kernel(plist_hbm, node_hbm, edge_hbm,
                      hij_hbm, hee_hbm,
                      pi_v, pj_v, nij_v, ee_v,
                      rows_ij, rows_ee, sem):
        wid = lax.axis_index("s") * NC + lax.axis_index("c")
        base = pl.multiple_of(wid * PW, PW)
        b_off = (wid // WPB) * N

        pltpu.sync_copy(plist_hbm.at[pl.ds(M + base, PW)], pi_v)
        pltpu.sync_copy(plist_hbm.at[pl.ds(2 * M + base, PW)], pj_v)

        lane2 = lax.broadcasted_iota(jnp.int32, (L,), 0) * 2

        def idx_body(k, carry):
            off = pl.multiple_of(k * L, L)
            i16 = pi_v[pl.ds(off, L)]
            j16 = pj_v[pl.ds(off, L)]
            ni = i16 + b_off
            nj = j16 + b_off
            ei = ni * N + j16
            idx2 = lane2 + 2 * off
            plsc.store_scatter(nij_v, [idx2], ni)
            plsc.store_scatter(nij_v, [idx2 + 1], nj)
            ee_v[pl.ds(off, L)] = ei
            return carry

        lax.fori_loop(0, PW // L, idx_body, 0)

        def gat_body(c, carry):
            r0 = pl.multiple_of(c * CH, CH)
            cp_ij = pltpu.async_copy(
                node_hbm.at[nij_v.at[pl.ds(2 * r0, 2 * CH)]], rows_ij, sem)
            cp_ee = pltpu.async_copy(
                edge_hbm.at[ee_v.at[pl.ds(r0, CH)]], rows_ee, sem)
            cp_ij.wait()
            cp_ee.wait()
            pltpu.sync_copy(rows_ij,
                            hij_hbm.at[pl.ds(2 * (base + r0), 2 * CH)])
            pltpu.sync_copy(rows_ee, hee_hbm.at[pl.ds(base + r0, CH)])
            return carry

        lax.fori_loop(0, NCH, gat_body, 0)

    return gather_kernel(plist_flat, node_tab, edge_tab)


def _tc_matmul(hij2, hee2, Wt, bias2d, M, H):
    BM = 2048
    OUT = Wt.shape[1]

    def mm_body(hij_ref, hee_ref, wt_ref, b_ref, o_ref):
        wt = wt_ref[...]
        hij = hij_ref[...]
        acc = jnp.dot(hij[:, 0:H], wt[0:H],
                      preferred_element_type=jnp.float32)
        acc = acc + jnp.dot(hij[:, H:2 * H], wt[H:2 * H],
                            preferred_element_type=jnp.float32)
        acc = acc + jnp.dot(hee_ref[:, 0:H], wt[2 * H:3 * H],
                            preferred_element_type=jnp.float32)
        o_ref[...] = acc + b_ref[...]

    return pl.pallas_call(
        mm_body,
        grid=(M // BM,),
        in_specs=[
            pl.BlockSpec((BM, 2 * H), lambda i: (i, 0)),
            pl.BlockSpec((BM, 2 * H), lambda i: (i, 0)),
            pl.BlockSpec((3 * H, OUT), lambda i: (0, 0)),
            pl.BlockSpec((1, OUT), lambda i: (0, 0)),
        ],
        out_specs=pl.BlockSpec((BM, OUT), lambda i: (i, 0)),
        out_shape=jax.ShapeDtypeStruct((M, OUT), jnp.float32),
    )(hij2, hee2, Wt, bias2d)


def kernel(h_node, h_edge, pairlist, W, bias):
    B, N, H = h_node.shape
    M = pairlist.shape[0]
    plist_flat = pairlist.T.reshape(-1)
    Wt = W.T
    bias2d = bias.reshape(1, -1)

    node_tab = h_node.reshape(B * N, H)
    edge_pad = jnp.concatenate(
        [jnp.pad(h_edge[b].reshape(N * N, H), ((0, 0), (0, H)))
         for b in range(B)], axis=0)
    hij, hee = _sc_gather(node_tab, edge_pad, plist_flat, B, N, H, M)
    hij2 = hij.reshape(M, 2 * H)
    out = _tc_matmul(hij2, hee, Wt, bias2d, M, H)
    return out.reshape(B, M // B, out.shape[-1])


# speedup vs baseline: 1.4996x; 1.4996x over previous
import functools

import jax
import jax.numpy as jnp
from jax import lax
from jax.experimental import pallas as pl
from jax.experimental.pallas import tpu as pltpu
from jax.experimental.pallas import tpu_sc as plsc

_SC_PARAMS = dict(use_tc_tiling_on_sc=False, needs_layout_passes=False)


def _sc_node_gather(node_tab, plist_flat, B, N, H, M):
    info = plsc.get_sparse_core_info()
    NC, NS, L = info.num_cores, info.num_subcores, info.num_lanes
    NW = NC * NS
    PW = M // NW
    CH = 64
    NCH = PW // CH
    WPB = NW // B

    mesh = plsc.VectorSubcoreMesh(core_axis_name="c", subcore_axis_name="s")

    @functools.partial(
        pl.kernel,
        mesh=mesh,
        compiler_params=pltpu.CompilerParams(**_SC_PARAMS),
        out_type=jax.ShapeDtypeStruct((2 * M, H), jnp.float32),
        scratch_types=[
            pltpu.VMEM((PW,), jnp.int32),
            pltpu.VMEM((PW,), jnp.int32),
            pltpu.VMEM((2 * PW,), jnp.int32),
            pltpu.VMEM((2 * CH, H), jnp.float32),
            pltpu.SemaphoreType.DMA,
        ],
    )
    def node_kernel(plist_hbm, node_hbm, hij_hbm,
                    pi_v, pj_v, nij_v, rows_ij, sem):
        wid = lax.axis_index("s") * NC + lax.axis_index("c")
        base = pl.multiple_of(wid * PW, PW)
        b_off = (wid // WPB) * N

        pltpu.sync_copy(plist_hbm.at[pl.ds(M + base, PW)], pi_v)
        pltpu.sync_copy(plist_hbm.at[pl.ds(2 * M + base, PW)], pj_v)

        lane2 = lax.broadcasted_iota(jnp.int32, (L,), 0) * 2

        def idx_body(k, carry):
            off = pl.multiple_of(k * L, L)
            idx2 = lane2 + 2 * off
            plsc.store_scatter(nij_v, [idx2], pi_v[pl.ds(off, L)] + b_off)
            plsc.store_scatter(nij_v, [idx2 + 1], pj_v[pl.ds(off, L)] + b_off)
            return carry

        lax.fori_loop(0, PW // L, idx_body, 0)

        def gat_body(c, carry):
            r0 = pl.multiple_of(c * CH, CH)
            cp = pltpu.async_copy(
                node_hbm.at[nij_v.at[pl.ds(2 * r0, 2 * CH)]], rows_ij, sem)
            cp.wait()
            pltpu.sync_copy(rows_ij,
                            hij_hbm.at[pl.ds(2 * (base + r0), 2 * CH)])
            return carry

        lax.fori_loop(0, NCH, gat_body, 0)

    return node_kernel(plist_flat, node_tab)


def _sc_edge_gather(edge_pad, plist_flat, m0, MC, B, N, H, M):
    info = plsc.get_sparse_core_info()
    NC, NS, L = info.num_cores, info.num_subcores, info.num_lanes
    NW = NC * NS
    PW = MC // NW
    CH = 128
    NCH = PW // CH
    MB = M // B
    WPB = MB // PW

    mesh = plsc.VectorSubcoreMesh(core_axis_name="c", subcore_axis_name="s")

    @functools.partial(
        pl.kernel,
        mesh=mesh,
        compiler_params=pltpu.CompilerParams(**_SC_PARAMS),
        out_type=jax.ShapeDtypeStruct((MC, 2 * H), jnp.float32),
        scratch_types=[
            pltpu.VMEM((PW,), jnp.int32),
            pltpu.VMEM((PW,), jnp.int32),
            pltpu.VMEM((PW,), jnp.int32),
            pltpu.VMEM((CH, 2 * H), jnp.float32),
            pltpu.SemaphoreType.DMA,
        ],
    )
    def edge_kernel(plist_hbm, edge_hbm, hee_hbm,
                    pi_v, pj_v, ee_v, rows_ee, sem):
        wid = lax.axis_index("s") * NC + lax.axis_index("c")
        base = pl.multiple_of(m0 + wid * PW, PW)
        b_off = (m0 // MB + wid // WPB) * N

        pltpu.sync_copy(plist_hbm.at[pl.ds(M + base, PW)], pi_v)
        pltpu.sync_copy(plist_hbm.at[pl.ds(2 * M + base, PW)], pj_v)

        def idx_body(k, carry):
            off = pl.multiple_of(k * L, L)
            j16 = pj_v[pl.ds(off, L)]
            ee_v[pl.ds(off, L)] = (pi_v[pl.ds(off, L)] + b_off) * N + j16
            return carry

        lax.fori_loop(0, PW // L, idx_body, 0)

        def gat_body(c, carry):
            r0 = pl.multiple_of(c * CH, CH)
            cp = pltpu.async_copy(
                edge_hbm.at[ee_v.at[pl.ds(r0, CH)]], rows_ee, sem)
            cp.wait()
            pltpu.sync_copy(rows_ee,
                            hee_hbm.at[pl.ds(base - m0 + r0, CH)])
            return carry

        lax.fori_loop(0, NCH, gat_body, 0)

    return edge_kernel(plist_flat, edge_pad)


def _tc_matmul(prev_out, hij2, hee_k, Wt, bias2d, m0, MC, M, H):
    BM = 2048
    OUT = Wt.shape[1]
    blk0 = m0 // BM

    def mm_body(*refs):
        hij_ref, hee_ref, wt_ref, b_ref, o_ref = refs[-5:]
        wt = wt_ref[...]
        hij = hij_ref[...]
        acc = jnp.dot(hij[:, 0:H], wt[0:H],
                      preferred_element_type=jnp.float32)
        acc = acc + jnp.dot(hij[:, H:2 * H], wt[H:2 * H],
                            preferred_element_type=jnp.float32)
        acc = acc + jnp.dot(hee_ref[:, 0:H], wt[2 * H:3 * H],
                            preferred_element_type=jnp.float32)
        o_ref[...] = acc + b_ref[...]

    in_specs = [
        pl.BlockSpec((BM, 2 * H), lambda i: (blk0 + i, 0)),
        pl.BlockSpec((BM, 2 * H), lambda i: (i, 0)),
        pl.BlockSpec((3 * H, OUT), lambda i: (0, 0)),
        pl.BlockSpec((1, OUT), lambda i: (0, 0)),
    ]
    args = (hij2, hee_k, Wt, bias2d)
    aliases = {}
    if prev_out is not None:
        in_specs = [pl.BlockSpec(memory_space=pl.ANY)] + in_specs
        args = (prev_out,) + args
        aliases = {0: 0}

    return pl.pallas_call(
        mm_body,
        grid=(MC // BM,),
        in_specs=in_specs,
        out_specs=pl.BlockSpec((BM, OUT), lambda i: (blk0 + i, 0)),
        out_shape=jax.ShapeDtypeStruct((M, OUT), jnp.float32),
        input_output_aliases=aliases,
    )(*args)


def kernel(h_node, h_edge, pairlist, W, bias):
    B, N, H = h_node.shape
    M = pairlist.shape[0]
    plist_flat = pairlist.T.reshape(-1)
    Wt = W.T
    bias2d = bias.reshape(1, -1)

    node_tab = h_node.reshape(B * N, H)
    edge_pad = jnp.pad(h_edge.reshape(B * N * N, H), ((0, 0), (0, H)))

    hij = _sc_node_gather(node_tab, plist_flat, B, N, H, M)
    hij2 = hij.reshape(M, 2 * H)

    MC = M // 2
    hee0 = _sc_edge_gather(edge_pad, plist_flat, 0, MC, B, N, H, M)
    hee1 = _sc_edge_gather(edge_pad, plist_flat, MC, MC, B, N, H, M)

    out = _tc_matmul(None, hij2, hee0, Wt, bias2d, 0, MC, M, H)
    out = _tc_matmul(out, hij2, hee1, Wt, bias2d, MC, MC, M, H)
    return out.reshape(B, M // B, out.shape[-1])
